# baseline (speedup 1.0000x reference)
import jax
import jax.numpy as jnp
from jax.experimental import pallas as pl
from jax.experimental.pallas import tpu as pltpu

_WINDOW = 32
_STEP = 16
_WIDTH = 4.0


def _body(x_ref, w1_ref, b1_ref, w2_hbm, b2_ref, o_ref, w2w_ref, sem):
    d = w1_ref.shape[1]
    ntok = o_ref.shape[1]
    half = _STEP * d
    bb = x_ref.shape[0]
    first = pl.program_id(0) == 0

    def _copy(k):
        return pltpu.make_async_copy(
            w2_hbm.at[pl.ds(k * half, half), :],
            w2w_ref.at[:, pl.ds(k * d, d)],
            sem.at[k],
        )

    @pl.when(first)
    def _start():
        _copy(0).start()
        _copy(1).start()

    w1q = w1_ref[...] * (1.0 / _WIDTH)
    b1q = b1_ref[0] * (1.0 / _WIDTH)

    def _stage1(i):
        ns = jnp.floor(
            jax.lax.dot_general(x_ref[i], w1q, (((0,), (0,)), ((), ())),
                                preferred_element_type=jnp.float32)
            + b1q)
        return (ns.astype(jnp.bfloat16).reshape(ns.shape[0] // _STEP, half)
                .astype(jnp.float32))

    def _stage2(i, chunks):
        cc = jnp.dot(chunks, w2w_ref[...], preferred_element_type=jnp.float32)
        o_ref[i] = jnp.floor(
            (cc[:ntok, :d] + cc[1 : ntok + 1, d:] + b2_ref[0]) * (1.0 / _WIDTH))

    prev = _stage1(0)
    cur = _stage1(1)

    @pl.when(first)
    def _wait():
        _copy(0).wait()
        _copy(1).wait()

    for i in range(2, bb):
        _stage2(i - 2, prev)
        prev, cur = cur, _stage1(i)
    _stage2(bb - 2, prev)
    _stage2(bb - 1, cur)


def kernel(x, W1, b1, W2, b2):
    batch, v, samples = x.shape
    d = W1.shape[1]
    num_tokens = (samples - _WINDOW) // _STEP
    b1r = b1.reshape(1, d)
    b2r = b2.reshape(1, d)
    bb = 8
    return pl.pallas_call(
        _body,
        grid=(batch // bb,),
        in_specs=[
            pl.BlockSpec((bb, v, samples), lambda b: (b, 0, 0)),
            pl.BlockSpec((v, d), lambda b: (0, 0)),
            pl.BlockSpec((1, d), lambda b: (0, 0)),
            pl.BlockSpec(memory_space=pltpu.MemorySpace.HBM),
            pl.BlockSpec((1, d), lambda b: (0, 0)),
        ],
        out_specs=pl.BlockSpec((bb, num_tokens, d), lambda b: (b, 0, 0)),
        out_shape=jax.ShapeDtypeStruct((batch, num_tokens, d), jnp.float32),
        scratch_shapes=[
            pltpu.VMEM((_STEP * d, 2 * d), jnp.float32),
            pltpu.SemaphoreType.DMA((2,)),
        ],
    )(x, W1, b1r, W2, b2r)
